# baseline (speedup 1.0000x reference)
import functools

import jax
import jax.numpy as jnp
from jax import lax
from jax.experimental import pallas as pl
from jax.experimental.pallas import tpu as pltpu
from jax.experimental.pallas import tpu_sc as plsc

VOCAB = 1000000
D = 64
B = 4096
S = 200

_INFO = plsc.get_sparse_core_info()
_NC, _NS = _INFO.num_cores, _INFO.num_subcores
_NW = _NC * _NS

_IW = 100
_NIDX = (B * S) // _IW
_IPW = _NIDX // _NW
_BPW = B // _NW
_NBG = 2
_KCH = _NBG * S // _IW
_NGRP = _BPW // _NBG
_HALF = _NGRP // 2


def _make_gather():
    mesh = plsc.VectorSubcoreMesh(core_axis_name="c", subcore_axis_name="s")

    @functools.partial(
        pl.kernel,
        mesh=mesh,
        out_type=jax.ShapeDtypeStruct((B, S, D), jnp.float32),
        scratch_types=[
            pltpu.VMEM((_IPW, _IW), jnp.int32),
            pltpu.VMEM((_NBG, S, D), jnp.float32),
            pltpu.VMEM((_NBG, S, D), jnp.float32),
            pltpu.SemaphoreType.DMA,
            pltpu.SemaphoreType.DMA,
            pltpu.SemaphoreType.DMA,
            pltpu.SemaphoreType.DMA,
        ],
        compiler_params=pltpu.CompilerParams(use_tc_tiling_on_sc=False),
    )
    def gather_kernel(table_hbm, idx_hbm, out_hbm, idx_v, rows0, rows1,
                      gs0, gs1, ws0, ws1):
        wid = lax.axis_index("s") * _NC + lax.axis_index("c")
        bbase = wid * _BPW
        pltpu.sync_copy(idx_hbm.at[pl.ds(wid * _IPW, _IPW)], idx_v)

        def fire_gathers(buf, sem, g):
            for j in range(_KCH):
                pltpu.async_copy(
                    table_hbm.at[idx_v.at[g * _KCH + j]],
                    buf.at[j // 2, pl.ds((j % 2) * _IW, _IW)],
                    sem)

        def wait_gathers(buf, sem):
            pltpu.make_async_copy(out_hbm.at[pl.ds(0, _NBG)], buf, sem).wait()

        def fire_write(buf, sem, g):
            pltpu.async_copy(buf, out_hbm.at[pl.ds(bbase + g * _NBG, _NBG)],
                             sem)

        def wait_write(buf, sem):
            pltpu.make_async_copy(buf, out_hbm.at[pl.ds(bbase, _NBG)],
                                  sem).wait()

        fire_gathers(rows0, gs0, 0)

        def body(t, carry):
            g0 = 2 * t
            wait_gathers(rows0, gs0)

            @pl.when(t >= 1)
            def _():
                wait_write(rows1, ws1)

            fire_write(rows0, ws0, g0)
            fire_gathers(rows1, gs1, g0 + 1)
            wait_gathers(rows1, gs1)
            wait_write(rows0, ws0)
            fire_write(rows1, ws1, g0 + 1)

            @pl.when(t <= _HALF - 2)
            def _():
                fire_gathers(rows0, gs0, g0 + 2)

            return carry

        lax.fori_loop(0, _HALF, body, 0)
        wait_write(rows1, ws1)

    return gather_kernel


_gather = _make_gather()


def kernel(tokens, embedding):
    idx = tokens.astype(jnp.int32).reshape(_NIDX, _IW)
    return _gather(embedding, idx)


# speedup vs baseline: 1.0054x; 1.0054x over previous
import functools

import jax
import jax.numpy as jnp
from jax import lax
from jax.experimental import pallas as pl
from jax.experimental.pallas import tpu as pltpu
from jax.experimental.pallas import tpu_sc as plsc

VOCAB = 1000000
D = 64
B = 4096
S = 200

_INFO = plsc.get_sparse_core_info()
_NC, _NS = _INFO.num_cores, _INFO.num_subcores
_NW = _NC * _NS

_N = B * S
_RPW = _N // _NW
_CW = 512
_NCH = _RPW // _CW
_HALF = _NCH // 2


def _make_gather():
    mesh = plsc.VectorSubcoreMesh(core_axis_name="c", subcore_axis_name="s")

    @functools.partial(
        pl.kernel,
        mesh=mesh,
        out_type=jax.ShapeDtypeStruct((_N, D), jnp.float32),
        scratch_types=[
            pltpu.VMEM((_RPW,), jnp.int32),
            pltpu.VMEM((_CW, D), jnp.float32),
            pltpu.VMEM((_CW, D), jnp.float32),
            pltpu.SemaphoreType.DMA,
            pltpu.SemaphoreType.DMA,
            pltpu.SemaphoreType.DMA,
            pltpu.SemaphoreType.DMA,
        ],
        compiler_params=pltpu.CompilerParams(use_tc_tiling_on_sc=False),
    )
    def gather_kernel(table_hbm, idx_hbm, out_hbm, idx_v, rows0, rows1,
                      gs0, gs1, ws0, ws1):
        wid = lax.axis_index("s") * _NC + lax.axis_index("c")
        rbase = wid * _RPW
        pltpu.sync_copy(idx_hbm.at[pl.ds(rbase, _RPW)], idx_v)

        def fire_gather(buf, sem, c):
            pltpu.async_copy(table_hbm.at[idx_v.at[pl.ds(c * _CW, _CW)]],
                             buf, sem)

        def wait_gather(buf, sem):
            pltpu.make_async_copy(out_hbm.at[pl.ds(0, _CW)], buf, sem).wait()

        def fire_write(buf, sem, c):
            pltpu.async_copy(buf, out_hbm.at[pl.ds(rbase + c * _CW, _CW)],
                             sem)

        def wait_write(buf, sem):
            pltpu.make_async_copy(buf, out_hbm.at[pl.ds(rbase, _CW)],
                                  sem).wait()

        fire_gather(rows0, gs0, 0)

        def body(t, carry):
            c0 = 2 * t
            wait_gather(rows0, gs0)

            @pl.when(t >= 1)
            def _():
                wait_write(rows1, ws1)

            fire_write(rows0, ws0, c0)
            fire_gather(rows1, gs1, c0 + 1)
            wait_gather(rows1, gs1)
            wait_write(rows0, ws0)
            fire_write(rows1, ws1, c0 + 1)

            @pl.when(t <= _HALF - 2)
            def _():
                fire_gather(rows0, gs0, c0 + 2)

            return carry

        lax.fori_loop(0, _HALF, body, 0)
        wait_write(rows1, ws1)

    return gather_kernel


_gather = _make_gather()


def kernel(tokens, embedding):
    idx = tokens.astype(jnp.int32).reshape(_N)
    out = _gather(embedding, idx)
    return out.reshape(B, S, D)
